# baseline (speedup 1.0000x reference)
import functools

import jax
import jax.numpy as jnp
from jax import lax
from jax.experimental import pallas as pl
from jax.experimental.pallas import tpu as pltpu
from jax.experimental.pallas import tpu_sc as plsc

N = 10000
E = 160000
R = 65
RW = R + 1
FIN = 128
H1 = 64
H2 = 32
B = 4096

NSC = 2
NTILE = 16
NW = NSC * NTILE

EPAD = 163840
PADE = EPAD - E
EPW = EPAD // NW
EPH = EPAD // NTILE
CH = 128

CNTSZ = 655360
ZB = 8192
AGGROWS = 10240
DP = 128
BPW = B // NW
NCH = EPW // CH
NCHH = EPH // CH
NROWS = EPAD // CH

@functools.cache
def _mesh():
    return plsc.VectorSubcoreMesh(
        core_axis_name="c", subcore_axis_name="s", num_cores=NSC,
        num_subcores=NTILE)


def _w_body(dstkR, etR, w_out, cnt_sp, zb, dstk2, et2, key2, cnt2, w2,
            ones_v, gsem, ssem):
    c = lax.axis_index("c")
    s = lax.axis_index("s")
    zs = CNTSZ // NTILE

    def fill0(i, carry):
        zb[pl.ds(i * 16, 16)] = jnp.zeros((16,), jnp.float32)
        return carry

    lax.fori_loop(0, ZB // 16, fill0, 0, unroll=8)
    for q in range(zs // ZB):
        pltpu.sync_copy(zb, cnt_sp.at[pl.ds(s * zs + q * ZB, ZB)])
    for i in range(CH // 16):
        ones_v[pl.ds(i * 16, 16)] = jnp.full((16,), 1.0, jnp.float32)
    pltpu.sync_copy(dstkR.at[pl.ds(s * NCHH, NCHH), :], dstk2)
    pltpu.sync_copy(etR.at[pl.ds(s * NCHH, NCHH), :], et2)

    def keys(k, carry):
        for i in range(CH // 16):
            sl = pl.ds(i * 16, 16)
            key2[k, sl] = dstk2[k, sl] * R + et2[k, sl]
        return carry

    lax.fori_loop(0, NCHH, keys, 0, unroll=2)
    plsc.subcore_barrier()

    def hist(g, carry):
        descs = [
            pltpu.async_copy(ones_v, cnt_sp.at[key2.at[g * 8 + j]], ssem,
                             add=True)
            for j in range(8)
        ]
        for dsc in descs:
            dsc.wait()
        return carry

    lax.fori_loop(0, NCHH // 8, hist, 0)
    plsc.subcore_barrier()

    row0 = c * (EPAD // NSC // CH) + s * NCH
    pltpu.sync_copy(dstkR.at[pl.ds(row0, NCH), :],
                    dstk2.at[pl.ds(0, NCH), :])
    pltpu.sync_copy(etR.at[pl.ds(row0, NCH), :], et2.at[pl.ds(0, NCH), :])
    lax.fori_loop(0, NCH, keys, 0, unroll=2)

    def wgather(g, carry):
        descs = [
            pltpu.async_copy(cnt_sp.at[key2.at[g * 8 + j]],
                             cnt2.at[g * 8 + j], gsem)
            for j in range(8)
        ]
        for dsc in descs:
            dsc.wait()
        return carry

    lax.fori_loop(0, NCH // 8, wgather, 0)

    def wcomp(k, carry):
        for i in range(CH // 16):
            sl = pl.ds(i * 16, 16)
            cc = jnp.maximum(cnt2[k, sl], jnp.full((16,), 1.0, jnp.float32))
            winv = jnp.full((16,), 1.0, jnp.float32) / cc
            pad = dstk2[k, sl] < jnp.full((16,), N, jnp.int32)
            w2[k, sl] = jnp.where(pad, winv, jnp.zeros((16,), jnp.float32))
        return carry

    lax.fori_loop(0, NCH, wcomp, 0, unroll=2)
    pltpu.sync_copy(w2, w_out.at[pl.ds(row0, NCH), :])


@functools.cache
def _w_kernel_built():
    return functools.partial(
        pl.kernel,
        out_type=jax.ShapeDtypeStruct((NROWS, CH), jnp.float32),
        mesh=_mesh(),
        scratch_types=[
            pltpu.VMEM_SHARED((CNTSZ,), jnp.float32),
            pltpu.VMEM((ZB,), jnp.float32),
            pltpu.VMEM((NCHH, CH), jnp.int32),
            pltpu.VMEM((NCHH, CH), jnp.int32),
            pltpu.VMEM((NCHH, CH), jnp.int32),
            pltpu.VMEM((NCH, CH), jnp.float32),
            pltpu.VMEM((NCH, CH), jnp.float32),
            pltpu.VMEM((CH,), jnp.float32),
            pltpu.SemaphoreType.DMA,
            pltpu.SemaphoreType.DMA,
        ],
    )(_w_body)


def _w_kernel(dstkR, etR):
    return _w_kernel_built()(dstkR, etR)


NCH2 = 8


def _gss_body(d, xw, srcR, etR, dstR, wR, aggp, agg_sp, gidx2, tmp2, dst2,
              w2, rows_a, rows_b, gsa, gsb, ssa):
    c = lax.axis_index("c")
    s = lax.axis_index("s")
    rpt = AGGROWS // NTILE
    nv = d // 16
    row0 = c * (EPAD // NSC // CH) + s * NCH

    def fill0(i, carry):
        for q in range(DP // 16):
            rows_a[i, pl.ds(q * 16, 16)] = jnp.zeros((16,), jnp.float32)
        return carry

    lax.fori_loop(0, CH, fill0, 0, unroll=8)
    for q in range(rpt // CH):
        sl = pl.ds(s * rpt + q * CH, CH)
        pltpu.sync_copy(rows_a, agg_sp.at[sl, :])

    one = jnp.full((16,), 1, jnp.int32)

    def keys(k, carry):
        for i in range(CH // 16):
            sl = pl.ds(i * 16, 16)
            gidx2[k, sl] = gidx2[k, sl] * RW + tmp2[k, sl] + one
        return carry

    def scale(kk, rows):
        def grp(g, carry):
            w16 = w2[kk, pl.ds(g * 16, 16)]
            for j in range(16):
                wb = jnp.broadcast_to(w16[j], (16,))
                r = g * 16 + j
                for q in range(nv):
                    sl = pl.ds(q * 16, 16)
                    rows[r, sl] = rows[r, sl] * wb
            return carry

        lax.fori_loop(0, CH // 16, grp, 0)

    plsc.subcore_barrier()
    for h in range(NCH // NCH2):
        r0 = row0 + h * NCH2
        pltpu.sync_copy(srcR.at[pl.ds(r0, NCH2), :], gidx2)
        pltpu.sync_copy(etR.at[pl.ds(r0, NCH2), :], tmp2)
        pltpu.sync_copy(dstR.at[pl.ds(r0, NCH2), :], dst2)
        pltpu.sync_copy(wR.at[pl.ds(r0, NCH2), :], w2)
        lax.fori_loop(0, NCH2, keys, 0, unroll=2)

        def pipe(k2, carry):
            k = 2 * k2
            da = pltpu.async_copy(xw.at[gidx2.at[k]], rows_a, gsa)
            db = pltpu.async_copy(xw.at[gidx2.at[k + 1]], rows_b, gsb)
            da.wait()
            scale(k, rows_a)
            sa = pltpu.async_copy(rows_a, agg_sp.at[dst2.at[k]], ssa,
                                  add=True)
            db.wait()
            scale(k + 1, rows_b)
            sa.wait()
            pltpu.sync_copy(rows_b, agg_sp.at[dst2.at[k + 1]], add=True)
            return carry

        lax.fori_loop(0, NCH2 // 2, pipe, 0)
    plsc.subcore_barrier()
    for q in range(rpt // CH):
        sl = pl.ds(s * rpt + q * CH, CH)
        pltpu.sync_copy(agg_sp.at[sl, :], rows_a)
        pltpu.sync_copy(rows_a, aggp.at[c, sl, :])


@functools.cache
def _gss_kernel(d):
    return functools.partial(
        pl.kernel,
        out_type=jax.ShapeDtypeStruct((NSC, AGGROWS, DP), jnp.float32),
        mesh=_mesh(),
        scratch_types=[
            pltpu.VMEM_SHARED((AGGROWS, DP), jnp.float32),
            pltpu.VMEM((NCH2, CH), jnp.int32),
            pltpu.VMEM((NCH2, CH), jnp.int32),
            pltpu.VMEM((NCH2, CH), jnp.int32),
            pltpu.VMEM((NCH2, CH), jnp.float32),
            pltpu.VMEM((CH, DP), jnp.float32),
            pltpu.VMEM((CH, DP), jnp.float32),
            pltpu.SemaphoreType.DMA,
            pltpu.SemaphoreType.DMA,
            pltpu.SemaphoreType.DMA,
        ],
    )(functools.partial(_gss_body, d))


def _gss64(xwflat, srcR, etR, dstR, wR):
    return _gss_kernel(H1)(xwflat, srcR, etR, dstR, wR)


def _gss32(xwflat, srcR, etR, dstR, wR):
    return _gss_kernel(H2)(xwflat, srcR, etR, dstR, wR)


EMBP = 256


def _mlpg_body(emb, f1, i0_hbm, i1_hbm, d1, d1o, d2, d2o, i0_v, i1_v, ba, bb,
               bc, bd, sem):
    c = lax.axis_index("c")
    s = lax.axis_index("s")
    base = (s * NSC + c) * BPW
    pltpu.sync_copy(i0_hbm.at[pl.ds(base, BPW)], i0_v)
    pltpu.sync_copy(i1_hbm.at[pl.ds(base, BPW)], i1_v)
    pltpu.async_copy(emb.at[i0_v], ba, sem).wait()
    pltpu.async_copy(f1.at[i0_v], bb, sem).wait()
    pltpu.async_copy(emb.at[i1_v], bc, sem).wait()
    pltpu.async_copy(f1.at[i1_v], bd, sem).wait()
    pltpu.sync_copy(ba, d1.at[pl.ds(base, BPW), :])
    pltpu.sync_copy(bb, d1o.at[pl.ds(base, BPW), :])
    pltpu.sync_copy(bc, d2.at[pl.ds(base, BPW), :])
    pltpu.sync_copy(bd, d2o.at[pl.ds(base, BPW), :])


@functools.cache
def _mlpg_built():
    return functools.partial(
        pl.kernel,
        out_type=(
            jax.ShapeDtypeStruct((B, EMBP), jnp.float32),
            jax.ShapeDtypeStruct((B, FIN), jnp.float32),
            jax.ShapeDtypeStruct((B, EMBP), jnp.float32),
            jax.ShapeDtypeStruct((B, FIN), jnp.float32),
        ),
        mesh=_mesh(),
        scratch_types=[
            pltpu.VMEM((BPW,), jnp.int32),
            pltpu.VMEM((BPW,), jnp.int32),
            pltpu.VMEM((BPW, EMBP), jnp.float32),
            pltpu.VMEM((BPW, FIN), jnp.float32),
            pltpu.VMEM((BPW, EMBP), jnp.float32),
            pltpu.VMEM((BPW, FIN), jnp.float32),
            pltpu.SemaphoreType.DMA,
        ],
    )(_mlpg_body)


def _mlpg_kernel(emb, f1, i0, i1):
    return _mlpg_built()(emb, f1, i0, i1)


def _mm_body(x_ref, w_ref, t_ref, r_ref):
    y = jnp.dot(x_ref[...], w_ref[...], preferred_element_type=jnp.float32)
    t_ref[...] = y.reshape(t_ref.shape)
    r_ref[...] = y[:, :DP]


def _mm(x, wbig, bn=400):
    x = x.astype(jnp.bfloat16)
    wbig = wbig.astype(jnp.bfloat16)
    n, k = x.shape
    m = wbig.shape[1]
    return pl.pallas_call(
        _mm_body,
        grid=(n // bn,),
        in_specs=[
            pl.BlockSpec((bn, k), lambda i: (i, 0)),
            pl.BlockSpec((k, m), lambda i: (0, 0)),
        ],
        out_specs=[
            pl.BlockSpec((bn * RW, DP), lambda i: (i, 0)),
            pl.BlockSpec((bn, DP), lambda i: (i, 0)),
        ],
        out_shape=[
            jax.ShapeDtypeStruct((n * RW, DP), jnp.float32),
            jax.ShapeDtypeStruct((n, DP), jnp.float32),
        ],
    )(x, wbig)


def _ep_body(d, relu, a_ref, xr_ref, b_ref, o_ref):
    v = a_ref[0, :, :d] + a_ref[1, :, :d] + xr_ref[:, :d] + b_ref[...]
    if relu:
        v = jnp.maximum(v, 0.0)
    o_ref[...] = v


def _epilogue(aggp, xroot, d, bvec, relu, bn=2000):
    return pl.pallas_call(
        functools.partial(_ep_body, d, relu),
        grid=(N // bn,),
        in_specs=[
            pl.BlockSpec((NSC, bn, DP), lambda i: (0, i, 0)),
            pl.BlockSpec((bn, DP), lambda i: (i, 0)),
            pl.BlockSpec((1, d), lambda i: (0, 0)),
        ],
        out_specs=pl.BlockSpec((bn, d), lambda i: (i, 0)),
        out_shape=jax.ShapeDtypeStruct((N, d), jnp.float32),
    )(aggp, xroot, bvec.reshape(1, d))


def _elu(v):
    return jnp.where(v > 0, v, jnp.exp(v) - 1.0)


def _mlp_body(d1, d1o, d2, d2o, wa, wb, wc, wd, b1m, w2, b2m, w3, b3m, o_ref):
    dot = functools.partial(jnp.dot, preferred_element_type=jnp.float32)
    h = (dot(d1[...], wa[...]) + dot(d1o[...], wb[...]) +
         dot(d2[...], wc[...]) + dot(d2o[...], wd[...]) + b1m[...])
    h = _elu(h)
    h = _elu(dot(h, w2[...]) + b2m[...])
    o_ref[...] = dot(h, w3[...]) + b3m[...]


def _mlp(d1, d1o, d2, d2o, w1, bv1, w2, bv2, w3p, bv3p, bb=512):
    wa = jnp.pad(w1[:160], ((0, EMBP - 160), (0, 0)))
    wb = w1[160:288]
    wc = jnp.pad(w1[288:448], ((0, EMBP - 160), (0, 0)))
    wd = w1[448:576]
    return pl.pallas_call(
        _mlp_body,
        grid=(B // bb,),
        in_specs=[
            pl.BlockSpec((bb, EMBP), lambda i: (i, 0)),
            pl.BlockSpec((bb, 128), lambda i: (i, 0)),
            pl.BlockSpec((bb, EMBP), lambda i: (i, 0)),
            pl.BlockSpec((bb, 128), lambda i: (i, 0)),
            pl.BlockSpec((EMBP, 256), lambda i: (0, 0)),
            pl.BlockSpec((128, 256), lambda i: (0, 0)),
            pl.BlockSpec((EMBP, 256), lambda i: (0, 0)),
            pl.BlockSpec((128, 256), lambda i: (0, 0)),
            pl.BlockSpec((1, 256), lambda i: (0, 0)),
            pl.BlockSpec((256, 128), lambda i: (0, 0)),
            pl.BlockSpec((1, 128), lambda i: (0, 0)),
            pl.BlockSpec((128, 128), lambda i: (0, 0)),
            pl.BlockSpec((1, 128), lambda i: (0, 0)),
        ],
        out_specs=pl.BlockSpec((bb, 128), lambda i: (i, 0)),
        out_shape=jax.ShapeDtypeStruct((B, 128), jnp.float32),
    )(d1, d1o, d2, d2o, wa, wb, wc, wd, bv1.reshape(1, 256), w2,
      bv2.reshape(1, 128), w3p, bv3p.reshape(1, 128))


def _prep_edges(ei, et):
    j = jnp.arange(PADE, dtype=jnp.int32)
    src = jnp.concatenate([ei[0], j % N]).reshape(NROWS, CH)
    dstk = jnp.concatenate([ei[1], N + (j % 64)]).reshape(NROWS, CH)
    dsts = jnp.concatenate([ei[1], j % N]).reshape(NROWS, CH)
    etp = jnp.concatenate([et, jnp.zeros((PADE,), jnp.int32)]).reshape(
        NROWS, CH)
    return src, dstk, dsts, etp


def _wbig(w_rel, w_root):
    w_all = jnp.concatenate([w_root[None], w_rel], axis=0)
    din, dout = w_all.shape[1], w_all.shape[2]
    w_pad = jnp.pad(w_all, ((0, 0), (0, 0), (0, DP - dout)))
    return w_pad.transpose(1, 0, 2).reshape(din, RW * DP)


def _branch(x, ei, et, w1r, w1root, bv1, w2r, w2root, bv2):
    src, dstk, dsts, etp = _prep_edges(ei, et)
    w = _w_kernel(dstk, etp)
    tab1, root1 = _mm(x, _wbig(w1r, w1root))
    aggp1 = _gss64(tab1, src, etp, dsts, w)
    h1 = _epilogue(aggp1, root1, H1, bv1, relu=True)
    tab2, root2 = _mm(h1, _wbig(w2r, w2root))
    aggp2 = _gss32(tab2, src, etp, dsts, w)
    h2 = _epilogue(aggp2, root2, H2, bv2, relu=False)
    return h2


def kernel(x_o, edge_index_o, edge_type_o, x_s0, edge_index_s0, edge_type_s0, x_s1, edge_index_s1, edge_type_s1, x_s2, edge_index_s2, edge_type_s2, x_s3, edge_index_s3, edge_type_s3, idx, W1_rel, W1_root, b1, W2_rel, W2_root, b2, Wo1_rel_0, Wo1_root_0, bo1_0, Wo2_rel_0, Wo2_root_0, bo2_0, Wo1_rel_1, Wo1_root_1, bo1_1, Wo2_rel_1, Wo2_root_1, bo2_1, Wo1_rel_2, Wo1_root_2, bo1_2, Wo2_rel_2, Wo2_root_2, bo2_2, Wo1_rel_3, Wo1_root_3, bo1_3, Wo2_rel_3, Wo2_root_3, bo2_3, features1, mlp1_w, mlp1_b, mlp2_w, mlp2_b, mlp3_w, mlp3_b):
    branches = [
        (x_o, edge_index_o, edge_type_o, W1_rel, W1_root, b1, W2_rel,
         W2_root, b2),
        (x_s0, edge_index_s0, edge_type_s0, Wo1_rel_0, Wo1_root_0, bo1_0,
         Wo2_rel_0, Wo2_root_0, bo2_0),
        (x_s1, edge_index_s1, edge_type_s1, Wo1_rel_1, Wo1_root_1, bo1_1,
         Wo2_rel_1, Wo2_root_1, bo2_1),
        (x_s2, edge_index_s2, edge_type_s2, Wo1_rel_2, Wo1_root_2, bo1_2,
         Wo2_rel_2, Wo2_root_2, bo2_2),
        (x_s3, edge_index_s3, edge_type_s3, Wo1_rel_3, Wo1_root_3, bo1_3,
         Wo2_rel_3, Wo2_root_3, bo2_3),
    ]
    h2s = [_branch(*br) for br in branches]
    embeds = jnp.concatenate(h2s, axis=1)
    emb_pad = jnp.concatenate(
        h2s + [jnp.zeros((N, EMBP - 160), jnp.float32)], axis=1)

    d1, d1o, d2, d2o = _mlpg_kernel(emb_pad, features1, idx[0], idx[1])
    w3p = jnp.zeros((128, 128), jnp.float32).at[:, :R].set(mlp3_w)
    b3p = jnp.zeros((128,), jnp.float32).at[:R].set(mlp3_b)
    mlp_out = _mlp(d1, d1o, d2, d2o, mlp1_w, mlp1_b, mlp2_w, mlp2_b, w3p,
                   b3p)[:, :R]
    return (embeds, mlp_out)


# speedup vs baseline: 1.0051x; 1.0051x over previous
import functools

import jax
import jax.numpy as jnp
from jax import lax
from jax.experimental import pallas as pl
from jax.experimental.pallas import tpu as pltpu
from jax.experimental.pallas import tpu_sc as plsc

N = 10000
E = 160000
R = 65
RW = R + 1
FIN = 128
H1 = 64
H2 = 32
B = 4096

NSC = 2
NTILE = 16
NW = NSC * NTILE

EPAD = 163840
PADE = EPAD - E
EPW = EPAD // NW
EPH = EPAD // NTILE
CH = 128

CNTSZ = 655360
ZB = 8192
AGGROWS = 10240
DP = 128
BPW = B // NW
NCH = EPW // CH
NCHH = EPH // CH
NROWS = EPAD // CH

@functools.cache
def _mesh():
    return plsc.VectorSubcoreMesh(
        core_axis_name="c", subcore_axis_name="s", num_cores=NSC,
        num_subcores=NTILE)


def _w_body(dstkR, etR, w_out, cnt_sp, zb, dstk2, et2, key2, cnt2, w2,
            ones_v, gsem, ssem):
    c = lax.axis_index("c")
    s = lax.axis_index("s")
    zs = CNTSZ // NTILE

    def fill0(i, carry):
        zb[pl.ds(i * 16, 16)] = jnp.zeros((16,), jnp.float32)
        return carry

    lax.fori_loop(0, ZB // 16, fill0, 0, unroll=8)
    for q in range(zs // ZB):
        pltpu.sync_copy(zb, cnt_sp.at[pl.ds(s * zs + q * ZB, ZB)])
    for i in range(CH // 16):
        ones_v[pl.ds(i * 16, 16)] = jnp.full((16,), 1.0, jnp.float32)
    pltpu.sync_copy(dstkR.at[pl.ds(s * NCHH, NCHH), :], dstk2)
    pltpu.sync_copy(etR.at[pl.ds(s * NCHH, NCHH), :], et2)

    def keys(k, carry):
        for i in range(CH // 16):
            sl = pl.ds(i * 16, 16)
            key2[k, sl] = dstk2[k, sl] * R + et2[k, sl]
        return carry

    lax.fori_loop(0, NCHH, keys, 0, unroll=2)
    plsc.subcore_barrier()

    def hist(g, carry):
        descs = [
            pltpu.async_copy(ones_v, cnt_sp.at[key2.at[g * 8 + j]], ssem,
                             add=True)
            for j in range(8)
        ]
        for dsc in descs:
            dsc.wait()
        return carry

    lax.fori_loop(0, NCHH // 8, hist, 0)
    plsc.subcore_barrier()

    row0 = c * (EPAD // NSC // CH) + s * NCH
    pltpu.sync_copy(dstkR.at[pl.ds(row0, NCH), :],
                    dstk2.at[pl.ds(0, NCH), :])
    pltpu.sync_copy(etR.at[pl.ds(row0, NCH), :], et2.at[pl.ds(0, NCH), :])
    lax.fori_loop(0, NCH, keys, 0, unroll=2)

    def wgather(g, carry):
        descs = [
            pltpu.async_copy(cnt_sp.at[key2.at[g * 8 + j]],
                             cnt2.at[g * 8 + j], gsem)
            for j in range(8)
        ]
        for dsc in descs:
            dsc.wait()
        return carry

    lax.fori_loop(0, NCH // 8, wgather, 0)

    def wcomp(k, carry):
        for i in range(CH // 16):
            sl = pl.ds(i * 16, 16)
            cc = jnp.maximum(cnt2[k, sl], jnp.full((16,), 1.0, jnp.float32))
            winv = jnp.full((16,), 1.0, jnp.float32) / cc
            pad = dstk2[k, sl] < jnp.full((16,), N, jnp.int32)
            w2[k, sl] = jnp.where(pad, winv, jnp.zeros((16,), jnp.float32))
        return carry

    lax.fori_loop(0, NCH, wcomp, 0, unroll=2)
    pltpu.sync_copy(w2, w_out.at[pl.ds(row0, NCH), :])


@functools.cache
def _w_kernel_built():
    return functools.partial(
        pl.kernel,
        out_type=jax.ShapeDtypeStruct((NROWS, CH), jnp.float32),
        mesh=_mesh(),
        scratch_types=[
            pltpu.VMEM_SHARED((CNTSZ,), jnp.float32),
            pltpu.VMEM((ZB,), jnp.float32),
            pltpu.VMEM((NCHH, CH), jnp.int32),
            pltpu.VMEM((NCHH, CH), jnp.int32),
            pltpu.VMEM((NCHH, CH), jnp.int32),
            pltpu.VMEM((NCH, CH), jnp.float32),
            pltpu.VMEM((NCH, CH), jnp.float32),
            pltpu.VMEM((CH,), jnp.float32),
            pltpu.SemaphoreType.DMA,
            pltpu.SemaphoreType.DMA,
        ],
    )(_w_body)


def _w_kernel(dstkR, etR):
    return _w_kernel_built()(dstkR, etR)


NCH2 = 8


def _gss_body(d, xw, srcR, etR, dstR, wR, aggp, agg_sp, gidx2, tmp2, dst2,
              w2, rows_a, rows_b, gsa, gsb, ssa):
    c = lax.axis_index("c")
    s = lax.axis_index("s")
    rpt = AGGROWS // NTILE
    nv = d // 16
    row0 = c * (EPAD // NSC // CH) + s * NCH

    def fill0(i, carry):
        for q in range(DP // 16):
            rows_a[i, pl.ds(q * 16, 16)] = jnp.zeros((16,), jnp.float32)
        return carry

    lax.fori_loop(0, CH, fill0, 0, unroll=8)
    for q in range(rpt // CH):
        sl = pl.ds(s * rpt + q * CH, CH)
        pltpu.sync_copy(rows_a, agg_sp.at[sl, :])

    one = jnp.full((16,), 1, jnp.int32)

    def keys(k, carry):
        for i in range(CH // 16):
            sl = pl.ds(i * 16, 16)
            gidx2[k, sl] = gidx2[k, sl] * RW + tmp2[k, sl] + one
        return carry

    def scale(kk, rows):
        def grp(g, carry):
            w16 = w2[kk, pl.ds(g * 16, 16)]
            for j in range(16):
                wb = jnp.broadcast_to(w16[j], (16,))
                r = g * 16 + j
                for q in range(nv):
                    sl = pl.ds(q * 16, 16)
                    rows[r, sl] = rows[r, sl] * wb
            return carry

        lax.fori_loop(0, CH // 16, grp, 0)

    plsc.subcore_barrier()
    for h in range(NCH // NCH2):
        r0 = row0 + h * NCH2
        pltpu.sync_copy(srcR.at[pl.ds(r0, NCH2), :], gidx2)
        pltpu.sync_copy(etR.at[pl.ds(r0, NCH2), :], tmp2)
        pltpu.sync_copy(dstR.at[pl.ds(r0, NCH2), :], dst2)
        pltpu.sync_copy(wR.at[pl.ds(r0, NCH2), :], w2)
        lax.fori_loop(0, NCH2, keys, 0, unroll=2)

        def pipe(k2, carry):
            k = 2 * k2
            da = pltpu.async_copy(xw.at[gidx2.at[k]], rows_a, gsa)
            db = pltpu.async_copy(xw.at[gidx2.at[k + 1]], rows_b, gsb)
            da.wait()
            scale(k, rows_a)
            sa = pltpu.async_copy(rows_a, agg_sp.at[dst2.at[k]], ssa,
                                  add=True)
            db.wait()
            scale(k + 1, rows_b)
            sa.wait()
            pltpu.sync_copy(rows_b, agg_sp.at[dst2.at[k + 1]], add=True)
            return carry

        lax.fori_loop(0, NCH2 // 2, pipe, 0)
    plsc.subcore_barrier()
    for q in range(rpt // CH):
        sl = pl.ds(s * rpt + q * CH, CH)
        pltpu.sync_copy(agg_sp.at[sl, :], rows_a)
        pltpu.sync_copy(rows_a, aggp.at[c, sl, :])


@functools.cache
def _gss_kernel(d):
    return functools.partial(
        pl.kernel,
        out_type=jax.ShapeDtypeStruct((NSC, AGGROWS, DP), jnp.float32),
        mesh=_mesh(),
        scratch_types=[
            pltpu.VMEM_SHARED((AGGROWS, DP), jnp.float32),
            pltpu.VMEM((NCH2, CH), jnp.int32),
            pltpu.VMEM((NCH2, CH), jnp.int32),
            pltpu.VMEM((NCH2, CH), jnp.int32),
            pltpu.VMEM((NCH2, CH), jnp.float32),
            pltpu.VMEM((CH, DP), jnp.float32),
            pltpu.VMEM((CH, DP), jnp.float32),
            pltpu.SemaphoreType.DMA,
            pltpu.SemaphoreType.DMA,
            pltpu.SemaphoreType.DMA,
        ],
    )(functools.partial(_gss_body, d))


def _gss64(xwflat, srcR, etR, dstR, wR):
    return _gss_kernel(H1)(xwflat, srcR, etR, dstR, wR)


def _gss32(xwflat, srcR, etR, dstR, wR):
    return _gss_kernel(H2)(xwflat, srcR, etR, dstR, wR)


EMBP = 256


def _mlpg_body(emb, f1, i0_hbm, i1_hbm, d1, d1o, d2, d2o, i0_v, i1_v, ba, bb,
               bc, bd, sem):
    c = lax.axis_index("c")
    s = lax.axis_index("s")
    base = (s * NSC + c) * BPW
    pltpu.sync_copy(i0_hbm.at[pl.ds(base, BPW)], i0_v)
    pltpu.sync_copy(i1_hbm.at[pl.ds(base, BPW)], i1_v)
    pltpu.async_copy(emb.at[i0_v], ba, sem).wait()
    pltpu.async_copy(f1.at[i0_v], bb, sem).wait()
    pltpu.async_copy(emb.at[i1_v], bc, sem).wait()
    pltpu.async_copy(f1.at[i1_v], bd, sem).wait()
    pltpu.sync_copy(ba, d1.at[pl.ds(base, BPW), :])
    pltpu.sync_copy(bb, d1o.at[pl.ds(base, BPW), :])
    pltpu.sync_copy(bc, d2.at[pl.ds(base, BPW), :])
    pltpu.sync_copy(bd, d2o.at[pl.ds(base, BPW), :])


@functools.cache
def _mlpg_built():
    return functools.partial(
        pl.kernel,
        out_type=(
            jax.ShapeDtypeStruct((B, EMBP), jnp.float32),
            jax.ShapeDtypeStruct((B, FIN), jnp.float32),
            jax.ShapeDtypeStruct((B, EMBP), jnp.float32),
            jax.ShapeDtypeStruct((B, FIN), jnp.float32),
        ),
        mesh=_mesh(),
        scratch_types=[
            pltpu.VMEM((BPW,), jnp.int32),
            pltpu.VMEM((BPW,), jnp.int32),
            pltpu.VMEM((BPW, EMBP), jnp.float32),
            pltpu.VMEM((BPW, FIN), jnp.float32),
            pltpu.VMEM((BPW, EMBP), jnp.float32),
            pltpu.VMEM((BPW, FIN), jnp.float32),
            pltpu.SemaphoreType.DMA,
        ],
    )(_mlpg_body)


def _mlpg_kernel(emb, f1, i0, i1):
    return _mlpg_built()(emb, f1, i0, i1)


def _mm_body(x_ref, w_ref, t_ref, r_ref):
    y = jnp.dot(x_ref[...], w_ref[...], preferred_element_type=jnp.float32)
    t_ref[...] = y.reshape(t_ref.shape)
    r_ref[...] = y[:, :DP]


def _mm(x, wbig, bn=400):
    n, k = x.shape
    m = wbig.shape[1]
    return pl.pallas_call(
        _mm_body,
        grid=(n // bn,),
        in_specs=[
            pl.BlockSpec((bn, k), lambda i: (i, 0)),
            pl.BlockSpec((k, m), lambda i: (0, 0)),
        ],
        out_specs=[
            pl.BlockSpec((bn * RW, DP), lambda i: (i, 0)),
            pl.BlockSpec((bn, DP), lambda i: (i, 0)),
        ],
        out_shape=[
            jax.ShapeDtypeStruct((n * RW, DP), jnp.float32),
            jax.ShapeDtypeStruct((n, DP), jnp.float32),
        ],
    )(x, wbig)


def _ep_body(d, relu, a_ref, xr_ref, b_ref, o_ref):
    v = a_ref[0, :, :d] + a_ref[1, :, :d] + xr_ref[:, :d] + b_ref[...]
    if relu:
        v = jnp.maximum(v, 0.0)
    o_ref[...] = v


def _epilogue(aggp, xroot, d, bvec, relu, bn=2000):
    return pl.pallas_call(
        functools.partial(_ep_body, d, relu),
        grid=(N // bn,),
        in_specs=[
            pl.BlockSpec((NSC, bn, DP), lambda i: (0, i, 0)),
            pl.BlockSpec((bn, DP), lambda i: (i, 0)),
            pl.BlockSpec((1, d), lambda i: (0, 0)),
        ],
        out_specs=pl.BlockSpec((bn, d), lambda i: (i, 0)),
        out_shape=jax.ShapeDtypeStruct((N, d), jnp.float32),
    )(aggp, xroot, bvec.reshape(1, d))


def _elu(v):
    return jnp.where(v > 0, v, jnp.exp(v) - 1.0)


def _mlp_body(d1, d1o, d2, d2o, wa, wb, wc, wd, b1m, w2, b2m, w3, b3m, o_ref):
    dot = functools.partial(jnp.dot, preferred_element_type=jnp.float32)
    h = (dot(d1[...], wa[...]) + dot(d1o[...], wb[...]) +
         dot(d2[...], wc[...]) + dot(d2o[...], wd[...]) + b1m[...])
    h = _elu(h)
    h = _elu(dot(h, w2[...]) + b2m[...])
    o_ref[...] = dot(h, w3[...]) + b3m[...]


def _mlp(d1, d1o, d2, d2o, w1, bv1, w2, bv2, w3p, bv3p, bb=512):
    wa = jnp.pad(w1[:160], ((0, EMBP - 160), (0, 0)))
    wb = w1[160:288]
    wc = jnp.pad(w1[288:448], ((0, EMBP - 160), (0, 0)))
    wd = w1[448:576]
    return pl.pallas_call(
        _mlp_body,
        grid=(B // bb,),
        in_specs=[
            pl.BlockSpec((bb, EMBP), lambda i: (i, 0)),
            pl.BlockSpec((bb, 128), lambda i: (i, 0)),
            pl.BlockSpec((bb, EMBP), lambda i: (i, 0)),
            pl.BlockSpec((bb, 128), lambda i: (i, 0)),
            pl.BlockSpec((EMBP, 256), lambda i: (0, 0)),
            pl.BlockSpec((128, 256), lambda i: (0, 0)),
            pl.BlockSpec((EMBP, 256), lambda i: (0, 0)),
            pl.BlockSpec((128, 256), lambda i: (0, 0)),
            pl.BlockSpec((1, 256), lambda i: (0, 0)),
            pl.BlockSpec((256, 128), lambda i: (0, 0)),
            pl.BlockSpec((1, 128), lambda i: (0, 0)),
            pl.BlockSpec((128, 128), lambda i: (0, 0)),
            pl.BlockSpec((1, 128), lambda i: (0, 0)),
        ],
        out_specs=pl.BlockSpec((bb, 128), lambda i: (i, 0)),
        out_shape=jax.ShapeDtypeStruct((B, 128), jnp.float32),
    )(d1, d1o, d2, d2o, wa, wb, wc, wd, bv1.reshape(1, 256), w2,
      bv2.reshape(1, 128), w3p, bv3p.reshape(1, 128))


def _prep_edges(ei, et):
    j = jnp.arange(PADE, dtype=jnp.int32)
    src = jnp.concatenate([ei[0], j % N]).reshape(NROWS, CH)
    dstk = jnp.concatenate([ei[1], N + (j % 64)]).reshape(NROWS, CH)
    dsts = jnp.concatenate([ei[1], j % N]).reshape(NROWS, CH)
    etp = jnp.concatenate([et, jnp.zeros((PADE,), jnp.int32)]).reshape(
        NROWS, CH)
    return src, dstk, dsts, etp


def _wbig(w_rel, w_root):
    w_all = jnp.concatenate([w_root[None], w_rel], axis=0)
    din, dout = w_all.shape[1], w_all.shape[2]
    w_pad = jnp.pad(w_all, ((0, 0), (0, 0), (0, DP - dout)))
    return w_pad.transpose(1, 0, 2).reshape(din, RW * DP)


def _branch(x, ei, et, w1r, w1root, bv1, w2r, w2root, bv2):
    src, dstk, dsts, etp = _prep_edges(ei, et)
    w = _w_kernel(dstk, etp)
    tab1, root1 = _mm(x, _wbig(w1r, w1root))
    aggp1 = _gss64(tab1, src, etp, dsts, w)
    h1 = _epilogue(aggp1, root1, H1, bv1, relu=True)
    tab2, root2 = _mm(h1, _wbig(w2r, w2root))
    aggp2 = _gss32(tab2, src, etp, dsts, w)
    h2 = _epilogue(aggp2, root2, H2, bv2, relu=False)
    return h2


def kernel(x_o, edge_index_o, edge_type_o, x_s0, edge_index_s0, edge_type_s0, x_s1, edge_index_s1, edge_type_s1, x_s2, edge_index_s2, edge_type_s2, x_s3, edge_index_s3, edge_type_s3, idx, W1_rel, W1_root, b1, W2_rel, W2_root, b2, Wo1_rel_0, Wo1_root_0, bo1_0, Wo2_rel_0, Wo2_root_0, bo2_0, Wo1_rel_1, Wo1_root_1, bo1_1, Wo2_rel_1, Wo2_root_1, bo2_1, Wo1_rel_2, Wo1_root_2, bo1_2, Wo2_rel_2, Wo2_root_2, bo2_2, Wo1_rel_3, Wo1_root_3, bo1_3, Wo2_rel_3, Wo2_root_3, bo2_3, features1, mlp1_w, mlp1_b, mlp2_w, mlp2_b, mlp3_w, mlp3_b):
    branches = [
        (x_o, edge_index_o, edge_type_o, W1_rel, W1_root, b1, W2_rel,
         W2_root, b2),
        (x_s0, edge_index_s0, edge_type_s0, Wo1_rel_0, Wo1_root_0, bo1_0,
         Wo2_rel_0, Wo2_root_0, bo2_0),
        (x_s1, edge_index_s1, edge_type_s1, Wo1_rel_1, Wo1_root_1, bo1_1,
         Wo2_rel_1, Wo2_root_1, bo2_1),
        (x_s2, edge_index_s2, edge_type_s2, Wo1_rel_2, Wo1_root_2, bo1_2,
         Wo2_rel_2, Wo2_root_2, bo2_2),
        (x_s3, edge_index_s3, edge_type_s3, Wo1_rel_3, Wo1_root_3, bo1_3,
         Wo2_rel_3, Wo2_root_3, bo2_3),
    ]
    h2s = [_branch(*br) for br in branches]
    embeds = jnp.concatenate(h2s, axis=1)
    emb_pad = jnp.concatenate(
        h2s + [jnp.zeros((N, EMBP - 160), jnp.float32)], axis=1)

    d1, d1o, d2, d2o = _mlpg_kernel(emb_pad, features1, idx[0], idx[1])
    w3p = jnp.zeros((128, 128), jnp.float32).at[:, :R].set(mlp3_w)
    b3p = jnp.zeros((128,), jnp.float32).at[:R].set(mlp3_b)
    mlp_out = _mlp(d1, d1o, d2, d2o, mlp1_w, mlp1_b, mlp2_w, mlp2_b, w3p,
                   b3p)[:, :R]
    return (embeds, mlp_out)
